# baseline (speedup 1.0000x reference)
import functools

import jax
import jax.numpy as jnp
from jax import lax
from jax.experimental import pallas as pl
from jax.experimental.pallas import tpu as pltpu
from jax.experimental.pallas import tpu_sc as plsc

N = 10000
E = 320000
D = 128
NC = 2
NS = 16
NW = NC * NS
EPW = E // NW
CH = 128
NCH = EPW // CH
TAIL = EPW - NCH * CH
NP = 10240
RPT = NP // NS

_sc_mesh = plsc.VectorSubcoreMesh(core_axis_name="c", subcore_axis_name="s")


@functools.partial(
    pl.kernel,
    out_type=jax.ShapeDtypeStruct((NC, NP), jnp.float32),
    mesh=_sc_mesh,
    compiler_params=pltpu.CompilerParams(needs_layout_passes=False),
    scratch_types=[
        pltpu.VMEM((EPW,), jnp.int32),
        pltpu.VMEM((NP,), jnp.float32),
        pltpu.VMEM((NS, RPT), jnp.float32),
        pltpu.VMEM((RPT,), jnp.float32),
        pltpu.VMEM_SHARED((NS, NP), jnp.float32),
    ],
)
def _deg_kernel(dst_hbm, out_hbm, dsti_v, hist_v, red_v, outv_v, sh):
    cid = lax.axis_index("c")
    sid = lax.axis_index("s")
    wid = cid * NS + sid
    pltpu.sync_copy(dst_hbm.at[pl.ds(wid * EPW, EPW)], dsti_v)

    def z(i, carry):
        hist_v[pl.ds(i * 16, 16)] = jnp.zeros((16,), jnp.float32)
        return carry

    lax.fori_loop(0, NP // 16, z, 0)

    ones16 = jnp.ones((16,), jnp.float32)

    def step(e, carry):
        idx16 = dsti_v[pl.ds(e * 16, 16)]
        plsc.addupdate_scatter(hist_v, [idx16], ones16)
        return carry

    lax.fori_loop(0, EPW // 16, step, 0)

    pltpu.sync_copy(hist_v, sh.at[sid])
    plsc.subcore_barrier()
    for j in range(NS):
        pltpu.sync_copy(sh.at[j, pl.ds(sid * RPT, RPT)], red_v.at[j])

    def rs(cb, carry):
        o = cb * 16
        s = red_v[0, pl.ds(o, 16)]
        for j in range(1, NS):
            s = s + red_v[j, pl.ds(o, 16)]
        outv_v[pl.ds(o, 16)] = s
        return carry

    lax.fori_loop(0, RPT // 16, rs, 0)
    pltpu.sync_copy(outv_v, out_hbm.at[cid, pl.ds(sid * RPT, RPT)])


@functools.partial(
    pl.kernel,
    out_type=jax.ShapeDtypeStruct((NC, NP, D), jnp.float32),
    mesh=_sc_mesh,
    scratch_types=[
        pltpu.VMEM((EPW,), jnp.int32),
        pltpu.VMEM((CH,), jnp.int32),
        pltpu.VMEM((CH,), jnp.int32),
        pltpu.VMEM((2, CH, D), jnp.float32),
        pltpu.VMEM((TAIL,), jnp.int32),
        pltpu.VMEM((TAIL, D), jnp.float32),
        pltpu.VMEM_SHARED((NP, D), jnp.float32),
        pltpu.SemaphoreType.DMA,
        pltpu.SemaphoreType.DMA,
        pltpu.SemaphoreType.DMA,
        pltpu.SemaphoreType.DMA,
        pltpu.SemaphoreType.DMA,
        pltpu.SemaphoreType.DMA,
    ],
)
def _msg_kernel(g_hbm, src_hbm, dst_hbm, zeros_hbm, out_hbm,
                srci_v, dst0_v, dst1_v, rows_v, taild_v, tailr_v, acc_sh,
                semr0, semr1, semi0, semi1, sems0, sems1):
    cid = lax.axis_index("c")
    sid = lax.axis_index("s")
    wid = cid * NS + sid
    pltpu.sync_copy(src_hbm.at[pl.ds(wid * EPW, EPW)], srci_v)
    pltpu.sync_copy(zeros_hbm.at[pl.ds(sid * RPT, RPT)],
                    acc_sh.at[pl.ds(sid * RPT, RPT)])
    plsc.subcore_barrier()

    semr = (semr0, semr1)
    semi = (semi0, semi1)
    sems = (sems0, sems1)
    dstb = (dst0_v, dst1_v)

    def fetch(c, b):
        dr = pltpu.async_copy(g_hbm.at[srci_v.at[pl.ds(c * CH, CH)]],
                              rows_v.at[b], semr[b])
        di = pltpu.async_copy(dst_hbm.at[pl.ds(wid * EPW + c * CH, CH)],
                              dstb[b], semi[b])
        return dr, di

    def wait_fetch(c, b):
        pltpu.make_async_copy(g_hbm.at[srci_v.at[pl.ds(c * CH, CH)]],
                              rows_v.at[b], semr[b]).wait()
        pltpu.make_async_copy(dst_hbm.at[pl.ds(wid * EPW + c * CH, CH)],
                              dstb[b], semi[b]).wait()

    def scatter(b):
        return pltpu.async_copy(rows_v.at[b], acc_sh.at[dstb[b]], sems[b],
                                add=True)

    def drain_scatter(b):
        pltpu.make_async_copy(rows_v.at[b], acc_sh.at[dstb[b]],
                              sems[b]).wait()

    pltpu.async_copy(g_hbm.at[srci_v.at[pl.ds(NCH * CH, TAIL)]],
                     tailr_v, semr0).wait()
    pltpu.async_copy(dst_hbm.at[pl.ds(wid * EPW + NCH * CH, TAIL)],
                     taild_v, semi0).wait()
    pltpu.sync_copy(tailr_v, acc_sh.at[taild_v], add=True)

    fetch(0, 0)
    fetch(1, 1)
    wait_fetch(0, 0)
    scatter(0)
    drain_scatter(0)
    fetch(2, 0)
    wait_fetch(1, 1)
    scatter(1)
    drain_scatter(1)
    fetch(3, 1)
    wait_fetch(2, 0)
    scatter(0)

    def pair(cc, carry):
        c1 = 2 * cc + 4
        drain_scatter(0)
        fetch(c1, 0)
        wait_fetch(c1 - 1, 1)
        scatter(1)
        c2 = c1 + 1
        drain_scatter(1)
        fetch(c2, 1)
        wait_fetch(c2 - 1, 0)
        scatter(0)
        return carry

    lax.fori_loop(0, (NCH - 4) // 2, pair, 0)
    wait_fetch(NCH - 1, 1)
    scatter(1)
    drain_scatter(0)
    drain_scatter(1)
    plsc.subcore_barrier()
    pltpu.sync_copy(acc_sh.at[pl.ds(sid * RPT, RPT)],
                    out_hbm.at[cid, pl.ds(sid * RPT, RPT)])


BN = 10000


def _dinv_from_parts(degp_ref):
    deg = degp_ref[0] + degp_ref[1] + 1.0
    return lax.rsqrt(deg)


def _gw_body(degp_ref, x_ref, w_ref, g_ref):
    h = jnp.dot(x_ref[...], w_ref[...], preferred_element_type=jnp.float32)
    g_ref[...] = h * _dinv_from_parts(degp_ref)


def _fin_body(acc_ref, g_ref, degp_ref, x_ref, b_ref, gam_ref, bet_ref, o_ref):
    dinv = _dinv_from_parts(degp_ref)
    s = (acc_ref[0] + acc_ref[1] + g_ref[...]) * dinv + b_ref[...]
    mean = jnp.mean(s, axis=-1, keepdims=True)
    c = s - mean
    var = jnp.mean(c * c, axis=-1, keepdims=True)
    y = c * lax.rsqrt(var + 1e-5) * gam_ref[...] + bet_ref[...]
    o_ref[...] = jnp.maximum(y, 0.0) + x_ref[...]


def _row_spec(w):
    return pl.BlockSpec((BN, w), lambda i: (i, 0))


def _part_spec(w):
    return pl.BlockSpec((NC, BN, w), lambda i: (0, i, 0))


def _full_spec(shape):
    return pl.BlockSpec(shape, lambda i: tuple(0 for _ in shape))


_gw_call = pl.pallas_call(
    _gw_body,
    grid=(N // BN,),
    in_specs=[_part_spec(1), _row_spec(D), _full_spec((D, D))],
    out_specs=_row_spec(D),
    out_shape=jax.ShapeDtypeStruct((N, D), jnp.float32),
)

_fin_call = pl.pallas_call(
    _fin_body,
    grid=(N // BN,),
    in_specs=[_part_spec(D), _row_spec(D), _part_spec(1), _row_spec(D),
              _full_spec((1, D)), _full_spec((1, D)), _full_spec((1, D))],
    out_specs=_row_spec(D),
    out_shape=jax.ShapeDtypeStruct((N, D), jnp.float32),
)


def kernel(x, edge_index, W, b, gamma, beta):
    src = edge_index[0].astype(jnp.int32)
    dst = edge_index[1].astype(jnp.int32)
    zeros_acc = jnp.zeros((NP, D), jnp.float32)

    degp = _deg_kernel(dst).reshape(NC, NP, 1)
    g = _gw_call(degp, x, W)
    accp = _msg_kernel(g, src, dst, zeros_acc)
    return _fin_call(accp, g, degp, x,
                     b.reshape(1, D), gamma.reshape(1, D), beta.reshape(1, D))


# speedup vs baseline: 1.0186x; 1.0186x over previous
import functools

import jax
import jax.numpy as jnp
from jax import lax
from jax.experimental import pallas as pl
from jax.experimental.pallas import tpu as pltpu
from jax.experimental.pallas import tpu_sc as plsc

N = 10000
E = 320000
D = 128
NC = 2
NS = 16
NW = NC * NS
EPW = E // NW
CH = 128
NCH = EPW // CH
TAIL = EPW - NCH * CH
NP = 10240
RPT = NP // NS

_sc_mesh = plsc.VectorSubcoreMesh(core_axis_name="c", subcore_axis_name="s")


@functools.partial(
    pl.kernel,
    out_type=jax.ShapeDtypeStruct((NC, NP), jnp.float32),
    mesh=_sc_mesh,
    compiler_params=pltpu.CompilerParams(needs_layout_passes=False),
    scratch_types=[
        pltpu.VMEM((EPW,), jnp.int32),
        pltpu.VMEM((NP,), jnp.float32),
        pltpu.VMEM((NS, RPT), jnp.float32),
        pltpu.VMEM((RPT,), jnp.float32),
        pltpu.VMEM_SHARED((NS, NP), jnp.float32),
    ],
)
def _deg_kernel(dst_hbm, out_hbm, dsti_v, hist_v, red_v, outv_v, sh):
    cid = lax.axis_index("c")
    sid = lax.axis_index("s")
    wid = cid * NS + sid
    pltpu.sync_copy(dst_hbm.at[pl.ds(wid * EPW, EPW)], dsti_v)

    def z(i, carry):
        hist_v[pl.ds(i * 16, 16)] = jnp.zeros((16,), jnp.float32)
        return carry

    lax.fori_loop(0, NP // 16, z, 0)

    ones16 = jnp.ones((16,), jnp.float32)

    def step(e, carry):
        idx16 = dsti_v[pl.ds(e * 16, 16)]
        plsc.addupdate_scatter(hist_v, [idx16], ones16)
        return carry

    lax.fori_loop(0, EPW // 16, step, 0)

    pltpu.sync_copy(hist_v, sh.at[sid])
    plsc.subcore_barrier()
    for j in range(NS):
        pltpu.sync_copy(sh.at[j, pl.ds(sid * RPT, RPT)], red_v.at[j])

    def rs(cb, carry):
        o = cb * 16
        s = red_v[0, pl.ds(o, 16)]
        for j in range(1, NS):
            s = s + red_v[j, pl.ds(o, 16)]
        outv_v[pl.ds(o, 16)] = s
        return carry

    lax.fori_loop(0, RPT // 16, rs, 0)
    pltpu.sync_copy(outv_v, out_hbm.at[cid, pl.ds(sid * RPT, RPT)])


@functools.partial(
    pl.kernel,
    out_type=jax.ShapeDtypeStruct((NC, NP, D), jnp.float32),
    mesh=_sc_mesh,
    scratch_types=[
        pltpu.VMEM((EPW,), jnp.int32),
        pltpu.VMEM((CH,), jnp.int32),
        pltpu.VMEM((CH,), jnp.int32),
        pltpu.VMEM((2, CH, D), jnp.float32),
        pltpu.VMEM((TAIL,), jnp.int32),
        pltpu.VMEM((TAIL, D), jnp.float32),
        pltpu.VMEM_SHARED((NP, D), jnp.float32),
        pltpu.SemaphoreType.DMA,
        pltpu.SemaphoreType.DMA,
        pltpu.SemaphoreType.DMA,
        pltpu.SemaphoreType.DMA,
        pltpu.SemaphoreType.DMA,
        pltpu.SemaphoreType.DMA,
    ],
)
def _msg_kernel(g_hbm, src_hbm, dst_hbm, zeros_hbm, out_hbm,
                srci_v, dst0_v, dst1_v, rows_v, taild_v, tailr_v, acc_sh,
                semr0, semr1, semi0, semi1, sems0, sems1):
    cid = lax.axis_index("c")
    sid = lax.axis_index("s")
    wid = cid * NS + sid
    pltpu.sync_copy(src_hbm.at[pl.ds(wid * EPW, EPW)], srci_v)
    pltpu.sync_copy(zeros_hbm.at[pl.ds(sid * RPT, RPT)],
                    acc_sh.at[pl.ds(sid * RPT, RPT)])
    plsc.subcore_barrier()

    semr = (semr0, semr1)
    semi = (semi0, semi1)
    sems = (sems0, sems1)
    dstb = (dst0_v, dst1_v)

    def fetch(c, b):
        dr = pltpu.async_copy(g_hbm.at[srci_v.at[pl.ds(c * CH, CH)]],
                              rows_v.at[b], semr[b])
        di = pltpu.async_copy(dst_hbm.at[pl.ds(wid * EPW + c * CH, CH)],
                              dstb[b], semi[b])
        return dr, di

    def wait_fetch(c, b):
        pltpu.make_async_copy(g_hbm.at[srci_v.at[pl.ds(c * CH, CH)]],
                              rows_v.at[b], semr[b]).wait()
        pltpu.make_async_copy(dst_hbm.at[pl.ds(wid * EPW + c * CH, CH)],
                              dstb[b], semi[b]).wait()

    def scatter(b):
        return pltpu.async_copy(rows_v.at[b], acc_sh.at[dstb[b]], sems[b],
                                add=True)

    def drain_scatter(b):
        pltpu.make_async_copy(rows_v.at[b], acc_sh.at[dstb[b]],
                              sems[b]).wait()

    pltpu.async_copy(g_hbm.at[srci_v.at[pl.ds(NCH * CH, TAIL)]],
                     tailr_v, semr0).wait()
    pltpu.async_copy(dst_hbm.at[pl.ds(wid * EPW + NCH * CH, TAIL)],
                     taild_v, semi0).wait()
    pltpu.sync_copy(tailr_v, acc_sh.at[taild_v], add=True)

    fetch(0, 0)
    fetch(1, 1)
    wait_fetch(0, 0)
    scatter(0)
    drain_scatter(0)
    fetch(2, 0)
    wait_fetch(1, 1)
    scatter(1)
    drain_scatter(1)
    fetch(3, 1)
    wait_fetch(2, 0)
    scatter(0)

    def pair(cc, carry):
        c1 = 2 * cc + 4
        drain_scatter(0)
        fetch(c1, 0)
        wait_fetch(c1 - 1, 1)
        scatter(1)
        c2 = c1 + 1
        drain_scatter(1)
        fetch(c2, 1)
        wait_fetch(c2 - 1, 0)
        scatter(0)
        return carry

    lax.fori_loop(0, (NCH - 4) // 2, pair, 0)
    wait_fetch(NCH - 1, 1)
    scatter(1)
    drain_scatter(0)
    drain_scatter(1)
    plsc.subcore_barrier()
    pltpu.sync_copy(acc_sh.at[pl.ds(sid * RPT, RPT)],
                    out_hbm.at[cid, pl.ds(sid * RPT, RPT)])


BN = 2000


def _dinv_from_parts(degp_ref):
    deg = degp_ref[0] + degp_ref[1] + 1.0
    return lax.rsqrt(deg)


def _gw_body(degp_ref, x_ref, w_ref, g_ref):
    h = jnp.dot(x_ref[...], w_ref[...], preferred_element_type=jnp.float32)
    g_ref[...] = h * _dinv_from_parts(degp_ref)


def _fin_body(acc_ref, g_ref, degp_ref, x_ref, b_ref, gam_ref, bet_ref, o_ref):
    dinv = _dinv_from_parts(degp_ref)
    s = (acc_ref[0] + acc_ref[1] + g_ref[...]) * dinv + b_ref[...]
    mean = jnp.mean(s, axis=-1, keepdims=True)
    c = s - mean
    var = jnp.mean(c * c, axis=-1, keepdims=True)
    y = c * lax.rsqrt(var + 1e-5) * gam_ref[...] + bet_ref[...]
    o_ref[...] = jnp.maximum(y, 0.0) + x_ref[...]


def _row_spec(w):
    return pl.BlockSpec((BN, w), lambda i: (i, 0))


def _part_spec(w):
    return pl.BlockSpec((NC, BN, w), lambda i: (0, i, 0))


def _full_spec(shape):
    return pl.BlockSpec(shape, lambda i: tuple(0 for _ in shape))


_gw_call = pl.pallas_call(
    _gw_body,
    grid=(N // BN,),
    in_specs=[_part_spec(1), _row_spec(D), _full_spec((D, D))],
    out_specs=_row_spec(D),
    out_shape=jax.ShapeDtypeStruct((N, D), jnp.float32),
)

_fin_call = pl.pallas_call(
    _fin_body,
    grid=(N // BN,),
    in_specs=[_part_spec(D), _row_spec(D), _part_spec(1), _row_spec(D),
              _full_spec((1, D)), _full_spec((1, D)), _full_spec((1, D))],
    out_specs=_row_spec(D),
    out_shape=jax.ShapeDtypeStruct((N, D), jnp.float32),
)


def kernel(x, edge_index, W, b, gamma, beta):
    src = edge_index[0].astype(jnp.int32)
    dst = edge_index[1].astype(jnp.int32)
    zeros_acc = jnp.zeros((NP, D), jnp.float32)

    degp = _deg_kernel(dst).reshape(NC, NP, 1)
    g = _gw_call(degp, x, W)
    accp = _msg_kernel(g, src, dst, zeros_acc)
    return _fin_call(accp, g, degp, x,
                     b.reshape(1, D), gamma.reshape(1, D), beta.reshape(1, D))
